# baseline (speedup 1.0000x reference)
import functools

import jax
import jax.numpy as jnp
from jax import lax
from jax.experimental import pallas as pl
from jax.experimental.pallas import tpu as pltpu
from jax.experimental.pallas import tpu_sc as plsc

N = 10000
E = 320000
D = 128
H = 128
C = 64
G = 64

NTILES = 16
NCORES = 2
CHUNK = 128
EPT = E // (NTILES * NCORES)
NCH = EPT // CHUNK
TAIL = EPT - NCH * CHUNK
NPAD = 10240
RPT = NPAD // NTILES
RB = RPT // CHUNK

_mesh = plsc.VectorSubcoreMesh(core_axis_name="c", subcore_axis_name="s")
_f32 = jnp.float32


def _zero_vmem_2d(ref, rows, cols):
    z16 = jnp.zeros((16,), _f32)
    def body(i, _):
        for l in range(cols // 16):
            ref[i, pl.ds(l * 16, 16)] = z16
        return 0
    lax.fori_loop(0, rows, body, 0)


@functools.partial(
    pl.kernel,
    out_type=jax.ShapeDtypeStruct((NCORES * NPAD,), _f32),
    mesh=_mesh,
    scratch_types=[
        pltpu.VMEM((CHUNK,), jnp.int32),
        pltpu.VMEM((CHUNK,), jnp.int32),
        pltpu.VMEM((TAIL,), jnp.int32),
        pltpu.VMEM((TAIL,), _f32),
        pltpu.VMEM((CHUNK,), _f32),
        pltpu.VMEM((RPT,), _f32),
        pltpu.VMEM_SHARED((NPAD,), _f32),
        pltpu.SemaphoreType.DMA,
        pltpu.SemaphoreType.DMA,
    ],
)
def _deg_kernel(dst_hbm, out_hbm, di_a, di_b, di_t, ones_t, ones_v, buf_v,
                acc_sh, sem_a, sem_b):
    c = lax.axis_index("c")
    s = lax.axis_index("s")
    wid = c * NTILES + s
    e0 = wid * EPT

    one16 = jnp.ones((16,), _f32)
    z16 = jnp.zeros((16,), _f32)
    for l in range(CHUNK // 16):
        ones_v[pl.ds(l * 16, 16)] = one16
    ones_t[...] = one16
    for l in range(RPT // 16):
        buf_v[pl.ds(l * 16, 16)] = z16

    pltpu.sync_copy(buf_v, acc_sh.at[pl.ds(s * RPT, RPT)])
    pltpu.async_copy(dst_hbm.at[pl.ds(e0, CHUNK)], di_a, sem_a)
    pltpu.async_copy(dst_hbm.at[pl.ds(e0 + CHUNK, CHUNK)], di_b, sem_b)
    plsc.subcore_barrier()

    def body(jj, _):
        j0 = 2 * jj
        pltpu.make_async_copy(
            dst_hbm.at[pl.ds(e0, CHUNK)], di_a, sem_a).wait()
        pltpu.sync_copy(ones_v, acc_sh.at[di_a], add=True)

        @pl.when(jj < NCH // 2 - 1)
        def _():
            pltpu.async_copy(
                dst_hbm.at[pl.ds(e0 + (j0 + 2) * CHUNK, CHUNK)],
                di_a, sem_a)

        pltpu.make_async_copy(
            dst_hbm.at[pl.ds(e0, CHUNK)], di_b, sem_b).wait()
        pltpu.sync_copy(ones_v, acc_sh.at[di_b], add=True)

        @pl.when(jj < NCH // 2 - 1)
        def _():
            pltpu.async_copy(
                dst_hbm.at[pl.ds(e0 + (j0 + 3) * CHUNK, CHUNK)],
                di_b, sem_b)
        return 0
    lax.fori_loop(0, NCH // 2, body, 0)

    pltpu.sync_copy(dst_hbm.at[pl.ds(e0 + NCH * CHUNK, TAIL)], di_t)
    pltpu.sync_copy(ones_t, acc_sh.at[di_t], add=True)

    plsc.subcore_barrier()
    pltpu.sync_copy(acc_sh.at[pl.ds(s * RPT, RPT)], buf_v)
    pltpu.sync_copy(buf_v, out_hbm.at[pl.ds(c * NPAD + s * RPT, RPT)])


@functools.partial(
    pl.kernel,
    out_type=jax.ShapeDtypeStruct((NCORES * NPAD, D), _f32),
    mesh=_mesh,
    scratch_types=[
        pltpu.VMEM((CHUNK,), jnp.int32),
        pltpu.VMEM((CHUNK,), jnp.int32),
        pltpu.VMEM((CHUNK,), jnp.int32),
        pltpu.VMEM((CHUNK,), jnp.int32),
        pltpu.VMEM((TAIL,), jnp.int32),
        pltpu.VMEM((TAIL,), jnp.int32),
        pltpu.VMEM((CHUNK, D), _f32),
        pltpu.VMEM((CHUNK, D), _f32),
        pltpu.VMEM((TAIL, D), _f32),
        pltpu.VMEM_SHARED((NPAD, D), _f32),
        pltpu.SemaphoreType.DMA,
        pltpu.SemaphoreType.DMA,
        pltpu.SemaphoreType.DMA,
        pltpu.SemaphoreType.DMA,
    ],
)
def _scatter_kernel(hs_hbm, src_hbm, dst_hbm, out_hbm,
                    si_a, si_b, di_a, di_b, si_t, di_t,
                    rows_a, rows_b, rows_t, acc_sh,
                    isem_a, isem_b, gsem_a, gsem_b):
    c = lax.axis_index("c")
    s = lax.axis_index("s")
    wid = c * NTILES + s
    e0 = wid * EPT

    def idx_load(buf_s, buf_d, j, sem):
        pltpu.async_copy(src_hbm.at[pl.ds(e0 + j * CHUNK, CHUNK)],
                         buf_s, sem)
        pltpu.async_copy(dst_hbm.at[pl.ds(e0 + j * CHUNK, CHUNK)],
                         buf_d, sem)

    def idx_wait(buf_s, buf_d, sem):
        pltpu.make_async_copy(src_hbm.at[pl.ds(e0, CHUNK)],
                              buf_s, sem).wait()
        pltpu.make_async_copy(dst_hbm.at[pl.ds(e0, CHUNK)],
                              buf_d, sem).wait()

    idx_load(si_a, di_a, 0, isem_a)
    idx_load(si_b, di_b, 1, isem_b)
    _zero_vmem_2d(rows_b, CHUNK, D)
    for k in range(RB):
        pltpu.sync_copy(rows_b, acc_sh.at[pl.ds(s * RPT + k * CHUNK, CHUNK)])
    idx_wait(si_a, di_a, isem_a)
    pltpu.async_copy(hs_hbm.at[si_a], rows_a, gsem_a)
    plsc.subcore_barrier()

    def body(jj, _):
        j0 = 2 * jj
        idx_wait(si_b, di_b, isem_b)
        pltpu.async_copy(hs_hbm.at[si_b], rows_b, gsem_b)
        pltpu.make_async_copy(hs_hbm.at[si_a], rows_a, gsem_a).wait()
        pltpu.sync_copy(rows_a, acc_sh.at[di_a], add=True)

        @pl.when(jj < NCH // 2 - 1)
        def _():
            idx_load(si_a, di_a, j0 + 2, isem_a)
            idx_wait(si_a, di_a, isem_a)
            pltpu.async_copy(hs_hbm.at[si_a], rows_a, gsem_a)

        pltpu.make_async_copy(hs_hbm.at[si_b], rows_b, gsem_b).wait()
        pltpu.sync_copy(rows_b, acc_sh.at[di_b], add=True)

        @pl.when(jj < NCH // 2 - 1)
        def _():
            idx_load(si_b, di_b, j0 + 3, isem_b)
        return 0
    lax.fori_loop(0, NCH // 2, body, 0)

    pltpu.sync_copy(src_hbm.at[pl.ds(e0 + NCH * CHUNK, TAIL)], si_t)
    pltpu.sync_copy(dst_hbm.at[pl.ds(e0 + NCH * CHUNK, TAIL)], di_t)
    pltpu.async_copy(hs_hbm.at[si_t], rows_t, gsem_a).wait()
    pltpu.sync_copy(rows_t, acc_sh.at[di_t], add=True)

    plsc.subcore_barrier()
    pltpu.sync_copy(acc_sh.at[pl.ds(s * RPT, RPT)],
                    out_hbm.at[pl.ds(c * NPAD + s * RPT, RPT)])


_PREC = lax.Precision.DEFAULT


def _mm_body(x_ref, w1_ref, h_ref):
    h_ref[...] = jnp.dot(x_ref[...], w1_ref[...],
                         preferred_element_type=_f32, precision=_PREC)


_tc_mm = pl.pallas_call(
    _mm_body,
    out_shape=jax.ShapeDtypeStruct((N, H), _f32),
)


def _scale_body(h_ref, degp_ref, hs_ref, dinv_ref):
    deg = degp_ref[0, :N] + degp_ref[1, :N] + 1.0
    dinv = lax.rsqrt(deg)[:, None]
    hs_ref[...] = h_ref[...] * dinv
    dinv_ref[...] = dinv


_tc_scale = pl.pallas_call(
    _scale_body,
    out_shape=(jax.ShapeDtypeStruct((N, H), _f32),
               jax.ShapeDtypeStruct((N, 1), _f32)),
)


def _norm_relu(z, g_ref, be_ref):
    mu = jnp.mean(z, axis=0, keepdims=True)
    var = jnp.mean(z * z, axis=0, keepdims=True) - mu * mu
    zn = g_ref[...] * (z - mu) * lax.rsqrt(var + 1e-5) + be_ref[...]
    return jnp.maximum(zn, 0.0)


def _combine(accp_ref, hs_ref, dinv_ref, b_ref):
    acc = accp_ref[0:N, :] + accp_ref[NPAD:NPAD + N, :]
    return (acc + hs_ref[...]) * dinv_ref[...] + b_ref[...]


def _tc2_body(accp_ref, hs1_ref, dinv_ref, b1_ref, g1_ref, be1_ref, w2_ref,
              hs2_ref):
    z = _combine(accp_ref, hs1_ref, dinv_ref, b1_ref)
    zn = _norm_relu(z, g1_ref, be1_ref)
    h2 = jnp.dot(zn, w2_ref[...], preferred_element_type=_f32, precision=_PREC)
    hs2_ref[...] = h2 * dinv_ref[...]


_tc2 = pl.pallas_call(
    _tc2_body,
    out_shape=jax.ShapeDtypeStruct((N, H), _f32),
)


def _tc3_body(accp_ref, hs2_ref, dinv_ref, b2_ref, g2_ref, be2_ref,
              batch_ref, wl_ref, bl_ref, out_ref):
    z = _combine(accp_ref, hs2_ref, dinv_ref, b2_ref)
    zn = _norm_relu(z, g2_ref, be2_ref)
    onehot = (batch_ref[...] ==
              lax.broadcasted_iota(jnp.int32, (N, G), 1)).astype(_f32)
    psum = lax.dot_general(onehot, zn, (((0,), (0,)), ((), ())),
                           preferred_element_type=_f32, precision=_PREC)
    cnt = jnp.sum(onehot, axis=0)[:, None]
    p = psum / jnp.maximum(cnt, 1.0)
    out_ref[...] = jnp.dot(p, wl_ref[...],
                           preferred_element_type=_f32, precision=_PREC) \
        + bl_ref[...]


_tc3 = pl.pallas_call(
    _tc3_body,
    out_shape=jax.ShapeDtypeStruct((G, C), _f32),
)


def kernel(x, edge_index, batch, W1, b1, g1, be1, W2, b2, g2, be2, Wl, bl):
    src = edge_index[0].astype(jnp.int32)
    dst = edge_index[1].astype(jnp.int32)

    degp = _deg_kernel(dst).reshape(NCORES, NPAD)

    h1 = _tc_mm(x, W1)
    hs1, dinv = _tc_scale(h1, degp)
    acc1 = _scatter_kernel(hs1, src, dst)
    hs2 = _tc2(acc1, hs1, dinv, b1.reshape(1, H), g1.reshape(1, H),
               be1.reshape(1, H), W2)
    acc2 = _scatter_kernel(hs2, src, dst)
    out = _tc3(acc2, hs2, dinv, b2.reshape(1, H), g2.reshape(1, H),
               be2.reshape(1, H), batch.astype(jnp.int32).reshape(N, 1),
               Wl, bl.reshape(1, C))
    return out


# speedup vs baseline: 1.1199x; 1.1199x over previous
import functools

import numpy as _np

import jax
import jax.numpy as jnp
from jax import lax
from jax.experimental import pallas as pl
from jax.experimental.pallas import tpu as pltpu
from jax.experimental.pallas import tpu_sc as plsc

N = 10000
E = 320000
D = 128
H = 128
C = 64
G = 64

NTILES = 16
NCORES = 2
CHUNK = 128
TPW = 80
EPAD = TPW * CHUNK * NTILES * NCORES
NPAD = 10240
RPT = NPAD // NTILES
RB = RPT // CHUNK

_mesh = plsc.VectorSubcoreMesh(core_axis_name="c", subcore_axis_name="s")
_f32 = jnp.float32


def _zero_vmem_2d(ref, rows, cols):
    z16 = jnp.zeros((16,), _f32)
    def body(i, _):
        for l in range(cols // 16):
            ref[i, pl.ds(l * 16, 16)] = z16
        return 0
    lax.fori_loop(0, rows, body, 0)


@functools.partial(
    pl.kernel,
    out_type=jax.ShapeDtypeStruct((NCORES * NPAD,), _f32),
    mesh=_mesh,
    scratch_types=[
        pltpu.VMEM((TPW, CHUNK), jnp.int32),
        pltpu.VMEM((CHUNK,), _f32),
        pltpu.VMEM((RPT,), _f32),
        pltpu.VMEM_SHARED((NPAD,), _f32),
    ],
)
def _deg_kernel(dst_hbm, out_hbm, dst_v, ones_v, buf_v, acc_sh):
    c = lax.axis_index("c")
    s = lax.axis_index("s")
    wid = c * NTILES + s

    one16 = jnp.ones((16,), _f32)
    z16 = jnp.zeros((16,), _f32)
    for l in range(CHUNK // 16):
        ones_v[pl.ds(l * 16, 16)] = one16
    for l in range(RPT // 16):
        buf_v[pl.ds(l * 16, 16)] = z16

    pltpu.sync_copy(buf_v, acc_sh.at[pl.ds(s * RPT, RPT)])
    pltpu.sync_copy(dst_hbm.at[pl.ds(wid * TPW, TPW)], dst_v)
    plsc.subcore_barrier()

    def body(j, _):
        pltpu.sync_copy(ones_v, acc_sh.at[dst_v.at[j]], add=True)
        return 0
    lax.fori_loop(0, TPW, body, 0)

    plsc.subcore_barrier()
    pltpu.sync_copy(acc_sh.at[pl.ds(s * RPT, RPT)], buf_v)
    pltpu.sync_copy(buf_v, out_hbm.at[pl.ds(c * NPAD + s * RPT, RPT)])


@functools.partial(
    pl.kernel,
    out_type=jax.ShapeDtypeStruct((NCORES * NPAD, D), _f32),
    mesh=_mesh,
    scratch_types=[
        pltpu.VMEM((TPW // 2, CHUNK), jnp.int32),
        pltpu.VMEM((TPW // 2, CHUNK), jnp.int32),
        pltpu.VMEM((CHUNK, D), _f32),
        pltpu.VMEM((CHUNK, D), _f32),
        pltpu.VMEM_SHARED((NPAD, D), _f32),
        pltpu.SemaphoreType.DMA,
        pltpu.SemaphoreType.DMA,
    ],
)
def _scatter_kernel(hs_hbm, src_hbm, dst_hbm, out_hbm,
                    src_v, dst_v, rows_a, rows_b, acc_sh, sem_a, sem_b):
    c = lax.axis_index("c")
    s = lax.axis_index("s")
    wid = c * NTILES + s
    half = TPW // 2

    _zero_vmem_2d(rows_b, CHUNK, D)
    for k in range(RB):
        pltpu.sync_copy(rows_b, acc_sh.at[pl.ds(s * RPT + k * CHUNK, CHUNK)])
    plsc.subcore_barrier()

    for h in range(2):
        r0 = wid * TPW + h * half
        pltpu.sync_copy(src_hbm.at[pl.ds(r0, half)], src_v)
        pltpu.sync_copy(dst_hbm.at[pl.ds(r0, half)], dst_v)
        pltpu.async_copy(hs_hbm.at[src_v.at[0]], rows_a, sem_a)

        def body(jj, _):
            j0 = 2 * jj
            pltpu.async_copy(hs_hbm.at[src_v.at[j0 + 1]], rows_b, sem_b)
            pltpu.make_async_copy(
                hs_hbm.at[src_v.at[j0]], rows_a, sem_a).wait()
            pltpu.sync_copy(rows_a, acc_sh.at[dst_v.at[j0]], add=True)

            @pl.when(jj < half // 2 - 1)
            def _():
                pltpu.async_copy(hs_hbm.at[src_v.at[j0 + 2]], rows_a, sem_a)

            pltpu.make_async_copy(
                hs_hbm.at[src_v.at[j0 + 1]], rows_b, sem_b).wait()
            pltpu.sync_copy(rows_b, acc_sh.at[dst_v.at[j0 + 1]], add=True)
            return 0
        lax.fori_loop(0, half // 2, body, 0)

    plsc.subcore_barrier()
    pltpu.sync_copy(acc_sh.at[pl.ds(s * RPT, RPT)],
                    out_hbm.at[pl.ds(c * NPAD + s * RPT, RPT)])


_PREC = lax.Precision.DEFAULT


def _mm_body(x_ref, w1_ref, h_ref):
    h_ref[...] = jnp.dot(x_ref[...], w1_ref[...],
                         preferred_element_type=_f32, precision=_PREC)


_tc_mm = pl.pallas_call(
    _mm_body,
    out_shape=jax.ShapeDtypeStruct((N, H), _f32),
)


def _scale_body(h_ref, degp_ref, hs_ref, dinv_ref):
    deg = degp_ref[0, :N] + degp_ref[1, :N] + 1.0
    dinv = lax.rsqrt(deg)[:, None]
    hs_ref[...] = h_ref[...] * dinv
    dinv_ref[...] = dinv


_tc_scale = pl.pallas_call(
    _scale_body,
    out_shape=(jax.ShapeDtypeStruct((N, H), _f32),
               jax.ShapeDtypeStruct((N, 1), _f32)),
)


def _norm_relu(z, g_ref, be_ref):
    mu = jnp.mean(z, axis=0, keepdims=True)
    var = jnp.mean(z * z, axis=0, keepdims=True) - mu * mu
    zn = g_ref[...] * (z - mu) * lax.rsqrt(var + 1e-5) + be_ref[...]
    return jnp.maximum(zn, 0.0)


def _combine(accp_ref, hs_ref, dinv_ref, b_ref):
    acc = accp_ref[0:N, :] + accp_ref[NPAD:NPAD + N, :]
    return (acc + hs_ref[...]) * dinv_ref[...] + b_ref[...]


def _tc2_body(accp_ref, hs1_ref, dinv_ref, b1_ref, g1_ref, be1_ref, w2_ref,
              hs2_ref):
    z = _combine(accp_ref, hs1_ref, dinv_ref, b1_ref)
    zn = _norm_relu(z, g1_ref, be1_ref)
    h2 = jnp.dot(zn, w2_ref[...], preferred_element_type=_f32, precision=_PREC)
    hs2_ref[...] = h2 * dinv_ref[...]


_tc2 = pl.pallas_call(
    _tc2_body,
    out_shape=jax.ShapeDtypeStruct((N, H), _f32),
)


def _tc3_body(accp_ref, hs2_ref, dinv_ref, b2_ref, g2_ref, be2_ref,
              batch_ref, wl_ref, bl_ref, out_ref):
    z = _combine(accp_ref, hs2_ref, dinv_ref, b2_ref)
    zn = _norm_relu(z, g2_ref, be2_ref)
    onehot = (batch_ref[...] ==
              lax.broadcasted_iota(jnp.int32, (N, G), 1)).astype(_f32)
    psum = lax.dot_general(onehot, zn, (((0,), (0,)), ((), ())),
                           preferred_element_type=_f32, precision=_PREC)
    cnt = jnp.sum(onehot, axis=0)[:, None]
    p = psum / jnp.maximum(cnt, 1.0)
    out_ref[...] = jnp.dot(p, wl_ref[...],
                           preferred_element_type=_f32, precision=_PREC) \
        + bl_ref[...]


_tc3 = pl.pallas_call(
    _tc3_body,
    out_shape=jax.ShapeDtypeStruct((G, C), _f32),
)


def kernel(x, edge_index, batch, W1, b1, g1, be1, W2, b2, g2, be2, Wl, bl):
    src = edge_index[0].astype(jnp.int32)
    dst = edge_index[1].astype(jnp.int32)
    pad = EPAD - E
    pad_ar = _np.arange(pad)
    src_tail = jnp.asarray(pad_ar % N, jnp.int32)
    dst_tail = jnp.asarray(N + pad_ar % (NPAD - N), jnp.int32)
    srcp = jnp.concatenate([src, src_tail]).reshape(EPAD // CHUNK, CHUNK)
    dstp = jnp.concatenate([dst, dst_tail]).reshape(EPAD // CHUNK, CHUNK)

    degp = _deg_kernel(dstp).reshape(NCORES, NPAD)

    h1 = _tc_mm(x, W1)
    hs1, dinv = _tc_scale(h1, degp)
    acc1 = _scatter_kernel(hs1, srcp, dstp)
    hs2 = _tc2(acc1, hs1, dinv, b1.reshape(1, H), g1.reshape(1, H),
               be1.reshape(1, H), W2)
    acc2 = _scatter_kernel(hs2, srcp, dstp)
    out = _tc3(acc2, hs2, dinv, b2.reshape(1, H), g2.reshape(1, H),
               be2.reshape(1, H), batch.astype(jnp.int32).reshape(N, 1),
               Wl, bl.reshape(1, C))
    return out


# speedup vs baseline: 1.1356x; 1.0140x over previous
import functools

import numpy as _np

import jax
import jax.numpy as jnp
from jax import lax
from jax.experimental import pallas as pl
from jax.experimental.pallas import tpu as pltpu
from jax.experimental.pallas import tpu_sc as plsc

N = 10000
E = 320000
D = 128
H = 128
C = 64
G = 64

NTILES = 16
NCORES = 2
CHUNK = 128
TPW = 80
EPAD = TPW * CHUNK * NTILES * NCORES
NPAD = 10240
RPT = NPAD // NTILES
RB = RPT // CHUNK

_mesh = plsc.VectorSubcoreMesh(core_axis_name="c", subcore_axis_name="s")
_f32 = jnp.float32


def _zero_vmem_2d(ref, rows, cols):
    z16 = jnp.zeros((16,), _f32)
    def body(i, _):
        for l in range(cols // 16):
            ref[i, pl.ds(l * 16, 16)] = z16
        return 0
    lax.fori_loop(0, rows, body, 0)


@functools.partial(
    pl.kernel,
    out_type=jax.ShapeDtypeStruct((NCORES * NPAD,), _f32),
    mesh=_mesh,
    scratch_types=[
        pltpu.VMEM((TPW, CHUNK), jnp.int32),
        pltpu.VMEM((CHUNK,), _f32),
        pltpu.VMEM((RPT,), _f32),
        pltpu.VMEM_SHARED((NPAD,), _f32),
        pltpu.SemaphoreType.DMA,
        pltpu.SemaphoreType.DMA,
    ],
)
def _deg_kernel(dst_hbm, out_hbm, dst_v, ones_v, buf_v, acc_sh,
                sem_a, sem_b):
    c = lax.axis_index("c")
    s = lax.axis_index("s")
    wid = c * NTILES + s

    one16 = jnp.ones((16,), _f32)
    z16 = jnp.zeros((16,), _f32)
    for l in range(CHUNK // 16):
        ones_v[pl.ds(l * 16, 16)] = one16
    for l in range(RPT // 16):
        buf_v[pl.ds(l * 16, 16)] = z16

    pltpu.sync_copy(buf_v, acc_sh.at[pl.ds(s * RPT, RPT)])
    pltpu.sync_copy(dst_hbm.at[pl.ds(wid * TPW, TPW)], dst_v)
    plsc.subcore_barrier()

    def body(jj, _):
        j0 = 2 * jj
        pltpu.async_copy(ones_v, acc_sh.at[dst_v.at[j0]], sem_a, add=True)
        pltpu.async_copy(ones_v, acc_sh.at[dst_v.at[j0 + 1]], sem_b,
                         add=True)
        pltpu.make_async_copy(ones_v, acc_sh.at[dst_v.at[j0]], sem_a).wait()
        pltpu.make_async_copy(ones_v, acc_sh.at[dst_v.at[j0 + 1]],
                              sem_b).wait()
        return 0
    lax.fori_loop(0, TPW // 2, body, 0)

    plsc.subcore_barrier()
    pltpu.sync_copy(acc_sh.at[pl.ds(s * RPT, RPT)], buf_v)
    pltpu.sync_copy(buf_v, out_hbm.at[pl.ds(c * NPAD + s * RPT, RPT)])


@functools.partial(
    pl.kernel,
    out_type=jax.ShapeDtypeStruct((NCORES * NPAD, D), _f32),
    mesh=_mesh,
    scratch_types=[
        pltpu.VMEM((TPW // 2, CHUNK), jnp.int32),
        pltpu.VMEM((TPW // 2, CHUNK), jnp.int32),
        pltpu.VMEM((CHUNK, D), _f32),
        pltpu.VMEM((CHUNK, D), _f32),
        pltpu.VMEM_SHARED((NPAD, D), _f32),
        pltpu.SemaphoreType.DMA,
        pltpu.SemaphoreType.DMA,
    ],
)
def _scatter_kernel(hs_hbm, src_hbm, dst_hbm, out_hbm,
                    src_v, dst_v, rows_a, rows_b, acc_sh, sem_a, sem_b):
    c = lax.axis_index("c")
    s = lax.axis_index("s")
    wid = c * NTILES + s
    half = TPW // 2

    _zero_vmem_2d(rows_b, CHUNK, D)
    for k in range(RB):
        pltpu.sync_copy(rows_b, acc_sh.at[pl.ds(s * RPT + k * CHUNK, CHUNK)])
    plsc.subcore_barrier()

    for h in range(2):
        r0 = wid * TPW + h * half
        pltpu.sync_copy(src_hbm.at[pl.ds(r0, half)], src_v)
        pltpu.sync_copy(dst_hbm.at[pl.ds(r0, half)], dst_v)
        pltpu.async_copy(hs_hbm.at[src_v.at[0]], rows_a, sem_a)

        def body(jj, _):
            j0 = 2 * jj
            pltpu.async_copy(hs_hbm.at[src_v.at[j0 + 1]], rows_b, sem_b)
            pltpu.make_async_copy(
                hs_hbm.at[src_v.at[j0]], rows_a, sem_a).wait()
            pltpu.sync_copy(rows_a, acc_sh.at[dst_v.at[j0]], add=True)

            @pl.when(jj < half // 2 - 1)
            def _():
                pltpu.async_copy(hs_hbm.at[src_v.at[j0 + 2]], rows_a, sem_a)

            pltpu.make_async_copy(
                hs_hbm.at[src_v.at[j0 + 1]], rows_b, sem_b).wait()
            pltpu.sync_copy(rows_b, acc_sh.at[dst_v.at[j0 + 1]], add=True)
            return 0
        lax.fori_loop(0, half // 2, body, 0)

    plsc.subcore_barrier()
    pltpu.sync_copy(acc_sh.at[pl.ds(s * RPT, RPT)],
                    out_hbm.at[pl.ds(c * NPAD + s * RPT, RPT)])


_PREC = lax.Precision.DEFAULT


def _mm_body(x_ref, w1_ref, h_ref):
    h_ref[...] = jnp.dot(x_ref[...], w1_ref[...],
                         preferred_element_type=_f32, precision=_PREC)


_tc_mm = pl.pallas_call(
    _mm_body,
    out_shape=jax.ShapeDtypeStruct((N, H), _f32),
)


def _scale_body(h_ref, degp_ref, hs_ref, dinv_ref):
    deg = degp_ref[0, :N] + degp_ref[1, :N] + 1.0
    dinv = lax.rsqrt(deg)[:, None]
    hs_ref[...] = h_ref[...] * dinv
    dinv_ref[...] = dinv


_tc_scale = pl.pallas_call(
    _scale_body,
    out_shape=(jax.ShapeDtypeStruct((N, H), _f32),
               jax.ShapeDtypeStruct((N, 1), _f32)),
)


def _norm_relu(z, g_ref, be_ref):
    mu = jnp.mean(z, axis=0, keepdims=True)
    var = jnp.mean(z * z, axis=0, keepdims=True) - mu * mu
    zn = g_ref[...] * (z - mu) * lax.rsqrt(var + 1e-5) + be_ref[...]
    return jnp.maximum(zn, 0.0)


def _combine(accp_ref, hs_ref, dinv_ref, b_ref):
    acc = accp_ref[0:N, :] + accp_ref[NPAD:NPAD + N, :]
    return (acc + hs_ref[...]) * dinv_ref[...] + b_ref[...]


def _tc2_body(accp_ref, hs1_ref, dinv_ref, b1_ref, g1_ref, be1_ref, w2_ref,
              hs2_ref):
    z = _combine(accp_ref, hs1_ref, dinv_ref, b1_ref)
    zn = _norm_relu(z, g1_ref, be1_ref)
    h2 = jnp.dot(zn, w2_ref[...], preferred_element_type=_f32, precision=_PREC)
    hs2_ref[...] = h2 * dinv_ref[...]


_tc2 = pl.pallas_call(
    _tc2_body,
    out_shape=jax.ShapeDtypeStruct((N, H), _f32),
)


def _tc3_body(accp_ref, hs2_ref, dinv_ref, b2_ref, g2_ref, be2_ref,
              batch_ref, wl_ref, bl_ref, out_ref):
    z = _combine(accp_ref, hs2_ref, dinv_ref, b2_ref)
    zn = _norm_relu(z, g2_ref, be2_ref)
    onehot = (batch_ref[...] ==
              lax.broadcasted_iota(jnp.int32, (N, G), 1)).astype(_f32)
    psum = lax.dot_general(onehot, zn, (((0,), (0,)), ((), ())),
                           preferred_element_type=_f32, precision=_PREC)
    cnt = jnp.sum(onehot, axis=0)[:, None]
    p = psum / jnp.maximum(cnt, 1.0)
    out_ref[...] = jnp.dot(p, wl_ref[...],
                           preferred_element_type=_f32, precision=_PREC) \
        + bl_ref[...]


_tc3 = pl.pallas_call(
    _tc3_body,
    out_shape=jax.ShapeDtypeStruct((G, C), _f32),
)


def kernel(x, edge_index, batch, W1, b1, g1, be1, W2, b2, g2, be2, Wl, bl):
    src = edge_index[0].astype(jnp.int32)
    dst = edge_index[1].astype(jnp.int32)
    pad = EPAD - E
    pad_ar = _np.arange(pad)
    src_tail = jnp.asarray(pad_ar % N, jnp.int32)
    dst_tail = jnp.asarray(N + pad_ar % (NPAD - N), jnp.int32)
    srcp = jnp.concatenate([src, src_tail]).reshape(EPAD // CHUNK, CHUNK)
    dstp = jnp.concatenate([dst, dst_tail]).reshape(EPAD // CHUNK, CHUNK)

    degp = _deg_kernel(dstp).reshape(NCORES, NPAD)

    h1 = _tc_mm(x, W1)
    hs1, dinv = _tc_scale(h1, degp)
    acc1 = _scatter_kernel(hs1, srcp, dstp)
    hs2 = _tc2(acc1, hs1, dinv, b1.reshape(1, H), g1.reshape(1, H),
               be1.reshape(1, H), W2)
    acc2 = _scatter_kernel(hs2, srcp, dstp)
    out = _tc3(acc2, hs2, dinv, b2.reshape(1, H), g2.reshape(1, H),
               be2.reshape(1, H), batch.astype(jnp.int32).reshape(N, 1),
               Wl, bl.reshape(1, C))
    return out


# speedup vs baseline: 1.1490x; 1.0118x over previous
import functools

import numpy as _np

import jax
import jax.numpy as jnp
from jax import lax
from jax.experimental import pallas as pl
from jax.experimental.pallas import tpu as pltpu
from jax.experimental.pallas import tpu_sc as plsc

N = 10000
E = 320000
D = 128
H = 128
C = 64
G = 64

NTILES = 16
NCORES = 2
CHUNK = 128
TPW = 80
EPAD = TPW * CHUNK * NTILES * NCORES
NPAD = 10240
RPT = NPAD // NTILES
RB = RPT // CHUNK

_mesh = plsc.VectorSubcoreMesh(core_axis_name="c", subcore_axis_name="s")
_f32 = jnp.float32


def _zero_vmem_2d(ref, rows, cols):
    z16 = jnp.zeros((16,), _f32)
    def body(i, _):
        for l in range(cols // 16):
            ref[i, pl.ds(l * 16, 16)] = z16
        return 0
    lax.fori_loop(0, rows, body, 0)


@functools.partial(
    pl.kernel,
    out_type=jax.ShapeDtypeStruct((NCORES * NPAD,), _f32),
    mesh=_mesh,
    scratch_types=[
        pltpu.VMEM((TPW, CHUNK), jnp.int32),
        pltpu.VMEM((CHUNK,), _f32),
        pltpu.VMEM((RPT,), _f32),
        pltpu.VMEM_SHARED((NPAD,), _f32),
        pltpu.SemaphoreType.DMA,
        pltpu.SemaphoreType.DMA,
    ],
)
def _deg_kernel(dst_hbm, out_hbm, dst_v, ones_v, buf_v, acc_sh,
                sem_a, sem_b):
    c = lax.axis_index("c")
    s = lax.axis_index("s")
    wid = c * NTILES + s

    one16 = jnp.ones((16,), _f32)
    z16 = jnp.zeros((16,), _f32)
    for l in range(CHUNK // 16):
        ones_v[pl.ds(l * 16, 16)] = one16
    for l in range(RPT // 16):
        buf_v[pl.ds(l * 16, 16)] = z16

    pltpu.sync_copy(buf_v, acc_sh.at[pl.ds(s * RPT, RPT)])
    pltpu.sync_copy(dst_hbm.at[pl.ds(wid * TPW, TPW)], dst_v)
    plsc.subcore_barrier()

    def body(jj, _):
        j0 = 2 * jj
        pltpu.async_copy(ones_v, acc_sh.at[dst_v.at[j0]], sem_a, add=True)
        pltpu.async_copy(ones_v, acc_sh.at[dst_v.at[j0 + 1]], sem_b,
                         add=True)
        pltpu.make_async_copy(ones_v, acc_sh.at[dst_v.at[j0]], sem_a).wait()
        pltpu.make_async_copy(ones_v, acc_sh.at[dst_v.at[j0 + 1]],
                              sem_b).wait()
        return 0
    lax.fori_loop(0, TPW // 2, body, 0)

    plsc.subcore_barrier()
    pltpu.sync_copy(acc_sh.at[pl.ds(s * RPT, RPT)], buf_v)
    pltpu.sync_copy(buf_v, out_hbm.at[pl.ds(c * NPAD + s * RPT, RPT)])


@functools.partial(
    pl.kernel,
    out_type=jax.ShapeDtypeStruct((NCORES * NPAD, D), _f32),
    mesh=_mesh,
    scratch_types=[
        pltpu.VMEM((TPW // 2, CHUNK), jnp.int32),
        pltpu.VMEM((TPW // 2, CHUNK), jnp.int32),
        pltpu.VMEM((CHUNK, D), _f32),
        pltpu.VMEM((CHUNK, D), _f32),
        pltpu.VMEM_SHARED((NPAD, D), _f32),
        pltpu.SemaphoreType.DMA,
        pltpu.SemaphoreType.DMA,
    ],
)
def _scatter_kernel(hs_hbm, src_hbm, dst_hbm, out_hbm,
                    src_v, dst_v, rows_a, rows_b, acc_sh, sem_a, sem_b):
    c = lax.axis_index("c")
    s = lax.axis_index("s")
    wid = c * NTILES + s
    half = TPW // 2

    pltpu.async_copy(src_hbm.at[pl.ds(wid * TPW, half)], src_v, sem_a)
    pltpu.async_copy(dst_hbm.at[pl.ds(wid * TPW, half)], dst_v, sem_b)
    _zero_vmem_2d(rows_b, CHUNK, D)
    for k in range(RB):
        pltpu.sync_copy(rows_b, acc_sh.at[pl.ds(s * RPT + k * CHUNK, CHUNK)])
    pltpu.make_async_copy(
        src_hbm.at[pl.ds(wid * TPW, half)], src_v, sem_a).wait()
    pltpu.make_async_copy(
        dst_hbm.at[pl.ds(wid * TPW, half)], dst_v, sem_b).wait()
    plsc.subcore_barrier()

    for h in range(2):
        if h == 1:
            r0 = wid * TPW + half
            pltpu.sync_copy(src_hbm.at[pl.ds(r0, half)], src_v)
            pltpu.sync_copy(dst_hbm.at[pl.ds(r0, half)], dst_v)
        pltpu.async_copy(hs_hbm.at[src_v.at[0]], rows_a, sem_a)

        def body(jj, _):
            j0 = 2 * jj
            pltpu.async_copy(hs_hbm.at[src_v.at[j0 + 1]], rows_b, sem_b)
            pltpu.make_async_copy(
                hs_hbm.at[src_v.at[j0]], rows_a, sem_a).wait()
            pltpu.sync_copy(rows_a, acc_sh.at[dst_v.at[j0]], add=True)

            @pl.when(jj < half // 2 - 1)
            def _():
                pltpu.async_copy(hs_hbm.at[src_v.at[j0 + 2]], rows_a, sem_a)

            pltpu.make_async_copy(
                hs_hbm.at[src_v.at[j0 + 1]], rows_b, sem_b).wait()
            pltpu.sync_copy(rows_b, acc_sh.at[dst_v.at[j0 + 1]], add=True)
            return 0
        lax.fori_loop(0, half // 2, body, 0)

    plsc.subcore_barrier()
    pltpu.sync_copy(acc_sh.at[pl.ds(s * RPT, RPT)],
                    out_hbm.at[pl.ds(c * NPAD + s * RPT, RPT)])


_PREC = lax.Precision.DEFAULT


def _mm_body(x_ref, w1_ref, h_ref):
    h_ref[...] = jnp.dot(x_ref[...], w1_ref[...],
                         preferred_element_type=_f32, precision=_PREC)


_tc_mm = pl.pallas_call(
    _mm_body,
    out_shape=jax.ShapeDtypeStruct((N, H), _f32),
)


def _scale_body(h_ref, degp_ref, hs_ref, dinv_ref):
    deg = degp_ref[0, :N] + degp_ref[1, :N] + 1.0
    dinv = lax.rsqrt(deg)[:, None]
    hs_ref[...] = h_ref[...] * dinv
    dinv_ref[...] = dinv


_tc_scale = pl.pallas_call(
    _scale_body,
    out_shape=(jax.ShapeDtypeStruct((N, H), _f32),
               jax.ShapeDtypeStruct((N, 1), _f32)),
)


def _norm_relu(z, g_ref, be_ref):
    mu = jnp.mean(z, axis=0, keepdims=True)
    var = jnp.mean(z * z, axis=0, keepdims=True) - mu * mu
    zn = g_ref[...] * (z - mu) * lax.rsqrt(var + 1e-5) + be_ref[...]
    return jnp.maximum(zn, 0.0)


def _combine(accp_ref, hs_ref, dinv_ref, b_ref):
    acc = accp_ref[0:N, :] + accp_ref[NPAD:NPAD + N, :]
    return (acc + hs_ref[...]) * dinv_ref[...] + b_ref[...]


def _tc2_body(accp_ref, hs1_ref, dinv_ref, b1_ref, g1_ref, be1_ref, w2_ref,
              hs2_ref):
    z = _combine(accp_ref, hs1_ref, dinv_ref, b1_ref)
    zn = _norm_relu(z, g1_ref, be1_ref)
    h2 = jnp.dot(zn, w2_ref[...], preferred_element_type=_f32, precision=_PREC)
    hs2_ref[...] = h2 * dinv_ref[...]


_tc2 = pl.pallas_call(
    _tc2_body,
    out_shape=jax.ShapeDtypeStruct((N, H), _f32),
)


def _tc3_body(accp_ref, hs2_ref, dinv_ref, b2_ref, g2_ref, be2_ref,
              batch_ref, wl_ref, bl_ref, out_ref):
    z = _combine(accp_ref, hs2_ref, dinv_ref, b2_ref)
    zn = _norm_relu(z, g2_ref, be2_ref)
    onehot = (batch_ref[...] ==
              lax.broadcasted_iota(jnp.int32, (N, G), 1)).astype(_f32)
    psum = lax.dot_general(onehot, zn, (((0,), (0,)), ((), ())),
                           preferred_element_type=_f32, precision=_PREC)
    cnt = jnp.sum(onehot, axis=0)[:, None]
    p = psum / jnp.maximum(cnt, 1.0)
    out_ref[...] = jnp.dot(p, wl_ref[...],
                           preferred_element_type=_f32, precision=_PREC) \
        + bl_ref[...]


_tc3 = pl.pallas_call(
    _tc3_body,
    out_shape=jax.ShapeDtypeStruct((G, C), _f32),
)


def kernel(x, edge_index, batch, W1, b1, g1, be1, W2, b2, g2, be2, Wl, bl):
    src = edge_index[0].astype(jnp.int32)
    dst = edge_index[1].astype(jnp.int32)
    pad = EPAD - E
    pad_ar = _np.arange(pad)
    src_tail = jnp.asarray(pad_ar % N, jnp.int32)
    dst_tail = jnp.asarray(N + pad_ar % (NPAD - N), jnp.int32)
    srcp = jnp.concatenate([src, src_tail]).reshape(EPAD // CHUNK, CHUNK)
    dstp = jnp.concatenate([dst, dst_tail]).reshape(EPAD // CHUNK, CHUNK)

    degp = _deg_kernel(dstp).reshape(NCORES, NPAD)

    h1 = _tc_mm(x, W1)
    hs1, dinv = _tc_scale(h1, degp)
    acc1 = _scatter_kernel(hs1, srcp, dstp)
    hs2 = _tc2(acc1, hs1, dinv, b1.reshape(1, H), g1.reshape(1, H),
               be1.reshape(1, H), W2)
    acc2 = _scatter_kernel(hs2, srcp, dstp)
    out = _tc3(acc2, hs2, dinv, b2.reshape(1, H), g2.reshape(1, H),
               be2.reshape(1, H), batch.astype(jnp.int32).reshape(N, 1),
               Wl, bl.reshape(1, C))
    return out
